# Initial kernel scaffold; baseline (speedup 1.0000x reference)
import jax
import jax.numpy as jnp
from jax.experimental import pallas as pl


def kernel(images, transforms):
    raise NotImplementedError("write your pallas kernel here")


# speedup vs baseline: 924.5731x; 924.5731x over previous
import functools

import jax
import jax.numpy as jnp
from jax import lax
from jax.experimental import pallas as pl
from jax.experimental.pallas import tpu as pltpu
from jax.experimental.pallas import tpu_sc as plsc

_L = 16
_NC = 2
_NS = 16
_NW = _NC * _NS
_PLANES = 96
_PIX = 512 * 512
_CH = 16384
_NCHUNK = _PIX // _CH
_PER_W = _PLANES // _NW
_TWO23 = 8388608.0


def _sc_lut_apply(images2d, luts2d):
    mesh = plsc.VectorSubcoreMesh(
        core_axis_name="c", subcore_axis_name="s",
        num_cores=_NC, num_subcores=_NS)

    @functools.partial(
        pl.kernel,
        mesh=mesh,
        compiler_params=pltpu.CompilerParams(needs_layout_passes=False),
        out_type=jax.ShapeDtypeStruct((_PLANES, _PIX), jnp.float32),
        scratch_types=[
            pltpu.VMEM((256,), jnp.float32),
            pltpu.VMEM((_CH,), jnp.float32),
            pltpu.VMEM((_CH,), jnp.float32),
        ],
    )
    def k(img_hbm, lut_hbm, out_hbm, lut_v, in_v, out_v):
        wid = lax.axis_index("s") * _NC + lax.axis_index("c")
        for j in range(_PER_W):
            p = wid * _PER_W + j
            pltpu.sync_copy(lut_hbm.at[p], lut_v)

            def chunk_body(c, _, p=p):
                pltpu.sync_copy(img_hbm.at[p, pl.ds(c * _CH, _CH)], in_v)

                def vec_body(i, _):
                    x = in_v[pl.ds(i * _L, _L)]
                    y = (x * 0.5 + 0.5) * 255.0
                    r = (y + _TWO23) - _TWO23
                    idx = r.astype(jnp.int32)
                    idx = jnp.minimum(jnp.maximum(idx, 0), 255)
                    out_v[pl.ds(i * _L, _L)] = plsc.load_gather(lut_v, [idx])
                    return 0

                lax.fori_loop(0, _CH // _L, vec_body, 0)
                pltpu.sync_copy(out_v, out_hbm.at[p, pl.ds(c * _CH, _CH)])
                return 0

            lax.fori_loop(0, _NCHUNK, chunk_body, 0)

    return k(images2d, luts2d)


def kernel(images, transforms):
    B, C, H, W = images.shape
    img2 = images.reshape(_PLANES, _PIX)
    lut2 = transforms.reshape(_PLANES, 256)
    out = _sc_lut_apply(img2, lut2)
    return out.reshape(B, C, H, W)
